# baseline (speedup 1.0000x reference)
import functools

import jax
import jax.numpy as jnp
from jax.experimental import pallas as pl

_K_HOPS = 3


def _tagconv(x, src, dst, edge_weight, W, b, k):
    N = x.shape[0]
    deg_dst = jnp.maximum(jnp.zeros((N,), x.dtype).at[dst].add(edge_weight), 1e-6)
    deg_src = jnp.maximum(jnp.zeros((N,), x.dtype).at[src].add(edge_weight), 1e-6)
    norm = edge_weight / jnp.sqrt(deg_src[src] * deg_dst[dst])
    fstack = [x]
    h = x
    for _ in range(k):
        msg = h[src] * norm[:, None]
        h = jnp.zeros((N, h.shape[1]), x.dtype).at[dst].add(msg)
        fstack.append(h)
    return jnp.concatenate(fstack, axis=-1) @ W + b


def _pred_body(z_ref, p1_ref, pb1_ref, p2_ref, pb2_ref, p3_ref, pb3_ref, o_ref):
    z = z_ref[...]
    t = z @ p1_ref[...] + pb1_ref[...]
    t = jnp.where(t > 0, t, 0.2 * t)
    t = t @ p2_ref[...] + pb2_ref[...]
    t = jnp.where(t > 0, t, 0.2 * t)
    o_ref[...] = t @ p3_ref[...] + pb3_ref[...]


def _predictor(z, P1, pb1, P2, pb2, P3, pb3):
    B = z.shape[0]
    blk = 400
    grid = (B // blk,)
    return pl.pallas_call(
        _pred_body,
        grid=grid,
        in_specs=[
            pl.BlockSpec((blk, 128), lambda i: (i, 0)),
            pl.BlockSpec((128, 64), lambda i: (0, 0)),
            pl.BlockSpec((64,), lambda i: (0,)),
            pl.BlockSpec((64, 32), lambda i: (0, 0)),
            pl.BlockSpec((32,), lambda i: (0,)),
            pl.BlockSpec((32, 1), lambda i: (0, 0)),
            pl.BlockSpec((1,), lambda i: (0,)),
        ],
        out_specs=pl.BlockSpec((blk, 1), lambda i: (i, 0)),
        out_shape=jax.ShapeDtypeStruct((B, 1), jnp.float32),
    )(z, P1, pb1, P2, pb2, P3, pb3)


def kernel(x, edge_index, edge_weight, pos_edges, neg_edges,
           W1, b1, W2, b2, W3, b3, P1, pb1, P2, pb2, P3, pb3):
    src, dst = edge_index[0], edge_index[1]
    h = _tagconv(x, src, dst, edge_weight, W1, b1, _K_HOPS)
    h = jax.nn.relu(h)
    h = _tagconv(h, src, dst, edge_weight, W2, b2, _K_HOPS)
    h = jax.nn.relu(h)
    h = _tagconv(h, src, dst, edge_weight, W3, b3, _K_HOPS)
    zp = h[pos_edges[0]] * h[pos_edges[1]]
    zn = h[neg_edges[0]] * h[neg_edges[1]]
    h_pos = _predictor(zp, P1, pb1, P2, pb2, P3, pb3)
    h_neg = _predictor(zn, P1, pb1, P2, pb2, P3, pb3)
    return (h_pos, h_neg, h)


# speedup vs baseline: 4.4038x; 4.4038x over previous
import functools

import jax
import jax.numpy as jnp
from jax import lax
from jax.experimental import pallas as pl
from jax.experimental.pallas import tpu as pltpu
from jax.experimental.pallas import tpu_sc as plsc

_NC, _NS, _L = 2, 16, 16
_NW = _NC * _NS
_D = 128
_DG = _D // _L


def _sc_mesh():
    return plsc.VectorSubcoreMesh(core_axis_name="c", subcore_axis_name="s")


def _rsqrt16(x):
    xi = plsc.bitcast(x, jnp.int32)
    yi = 0x5F3759DF - lax.shift_right_arithmetic(xi, 1)
    y = plsc.bitcast(yi, jnp.float32)
    for _ in range(3):
        y = y * (1.5 - 0.5 * x * y * y)
    return y


def _deg_partials(src, dst, w, n_pad):
    E = src.shape[0]
    e_per = E // _NW

    @functools.partial(
        pl.kernel,
        compiler_params=pltpu.CompilerParams(needs_layout_passes=False),
        out_type=jax.ShapeDtypeStruct((_NW, 2, n_pad), jnp.float32),
        mesh=_sc_mesh(),
        scratch_types=[
            pltpu.VMEM((e_per,), jnp.int32),
            pltpu.VMEM((e_per,), jnp.int32),
            pltpu.VMEM((e_per,), jnp.float32),
            pltpu.VMEM((n_pad,), jnp.float32),
            pltpu.VMEM((n_pad,), jnp.float32),
        ],
    )
    def k(src_h, dst_h, w_h, out_h, src_v, dst_v, w_v, ds_v, dd_v):
        c = lax.axis_index("c")
        s = lax.axis_index("s")
        wid = c * _NS + s
        base = wid * e_per
        z = jnp.zeros((_L,), jnp.float32)

        def zero(i, carry):
            ds_v[pl.ds(i * _L, _L)] = z
            dd_v[pl.ds(i * _L, _L)] = z
            return carry

        lax.fori_loop(0, n_pad // _L, zero, 0)
        pltpu.sync_copy(src_h.at[pl.ds(base, e_per)], src_v)
        pltpu.sync_copy(dst_h.at[pl.ds(base, e_per)], dst_v)
        pltpu.sync_copy(w_h.at[pl.ds(base, e_per)], w_v)

        def body(i, carry):
            sl = pl.ds(i * _L, _L)
            wv = w_v[sl]
            plsc.addupdate_scatter(ds_v, [src_v[sl]], wv)
            plsc.addupdate_scatter(dd_v, [dst_v[sl]], wv)
            return carry

        lax.fori_loop(0, e_per // _L, body, 0)
        pltpu.sync_copy(ds_v, out_h.at[wid, 0])
        pltpu.sync_copy(dd_v, out_h.at[wid, 1])

    return k(src, dst, w)


def _edge_norm(parts, src, dst, w, n_pad):
    E = src.shape[0]
    e_per = E // _NW
    npc = n_pad // _NS
    CH = 2000

    @functools.partial(
        pl.kernel,
        compiler_params=pltpu.CompilerParams(needs_layout_passes=False),
        out_type=jax.ShapeDtypeStruct((E,), jnp.float32),
        mesh=_sc_mesh(),
        scratch_types=[
            pltpu.VMEM((_NW, 2, npc), jnp.float32),
            pltpu.VMEM((2, npc), jnp.float32),
            pltpu.VMEM((n_pad,), jnp.float32),
            pltpu.VMEM((n_pad,), jnp.float32),
            pltpu.VMEM_SHARED((2, n_pad), jnp.float32),
            pltpu.VMEM((CH,), jnp.int32),
            pltpu.VMEM((CH,), jnp.int32),
            pltpu.VMEM((CH,), jnp.float32),
            pltpu.VMEM((CH,), jnp.float32),
        ],
    )
    def k(parts_h, src_h, dst_h, w_h, norm_h, stage_v, rsl_v, rss_v, rsd_v,
          rs_sh, src_v, dst_v, w_v, nrm_v):
        c = lax.axis_index("c")
        s = lax.axis_index("s")
        wid = c * _NS + s
        nbase = s * npc

        def ldp(p, carry):
            pltpu.sync_copy(parts_h.at[p, 0, pl.ds(nbase, npc)], stage_v.at[p, 0])
            pltpu.sync_copy(parts_h.at[p, 1, pl.ds(nbase, npc)], stage_v.at[p, 1])
            return carry

        lax.fori_loop(0, _NW, ldp, 0)

        def red(i, carry):
            sl = pl.ds(i * _L, _L)

            def acc(p, ab):
                return (ab[0] + stage_v[p, 0, sl], ab[1] + stage_v[p, 1, sl])

            zz = jnp.zeros((_L,), jnp.float32)
            a, b = lax.fori_loop(0, _NW, acc, (zz, zz))
            rsl_v[0, sl] = _rsqrt16(jnp.maximum(a, 1e-6))
            rsl_v[1, sl] = _rsqrt16(jnp.maximum(b, 1e-6))
            return carry

        lax.fori_loop(0, npc // _L, red, 0)
        pltpu.sync_copy(rsl_v.at[0], rs_sh.at[0, pl.ds(nbase, npc)])
        pltpu.sync_copy(rsl_v.at[1], rs_sh.at[1, pl.ds(nbase, npc)])
        plsc.subcore_barrier()
        pltpu.sync_copy(rs_sh.at[0], rss_v)
        pltpu.sync_copy(rs_sh.at[1], rsd_v)

        ebase = wid * e_per

        def chunk(j, carry):
            cb = ebase + j * CH
            pltpu.sync_copy(src_h.at[pl.ds(cb, CH)], src_v)
            pltpu.sync_copy(dst_h.at[pl.ds(cb, CH)], dst_v)
            pltpu.sync_copy(w_h.at[pl.ds(cb, CH)], w_v)

            def inner(i, carry2):
                sl = pl.ds(i * _L, _L)
                a = plsc.load_gather(rss_v, [src_v[sl]])
                b = plsc.load_gather(rsd_v, [dst_v[sl]])
                nrm_v[sl] = w_v[sl] * a * b
                return carry2

            lax.fori_loop(0, CH // _L, inner, 0)
            pltpu.sync_copy(nrm_v, norm_h.at[pl.ds(cb, CH)])
            return carry

        lax.fori_loop(0, e_per // CH, chunk, 0)

    return k(parts, src, dst, w)


def _prop(h, src, dst, norm):
    N = h.shape[0]
    E = src.shape[0]
    e_per = E // _NW
    C = 80
    n_chunks = e_per // C
    rpt = N // _NS
    ZR = 128

    @functools.partial(
        pl.kernel,
        compiler_params=pltpu.CompilerParams(needs_layout_passes=False),
        out_type=jax.ShapeDtypeStruct((_NC, N, _D), jnp.float32),
        mesh=_sc_mesh(),
        scratch_types=[
            pltpu.VMEM_SHARED((N, _D), jnp.float32),
            pltpu.VMEM((C,), jnp.int32),
            pltpu.VMEM((C,), jnp.int32),
            pltpu.VMEM((C,), jnp.float32),
            pltpu.VMEM((C, _D), jnp.float32),
            pltpu.VMEM((ZR, _D), jnp.float32),
            pltpu.SemaphoreType.DMA,
        ],
    )
    def k(h_h, src_h, dst_h, nrm_h, out_h, acc_sh, src_v, dst_v, nrm_v,
          rows_v, zb_v, sem):
        c = lax.axis_index("c")
        s = lax.axis_index("s")
        wid = c * _NS + s
        z = jnp.zeros((_L,), jnp.float32)

        def zb(r, carry):
            for j in range(_DG):
                zb_v[r, pl.ds(j * _L, _L)] = z
            return carry

        lax.fori_loop(0, ZR, zb, 0)
        rbase = s * rpt

        def zc(i, carry):
            pltpu.sync_copy(zb_v, acc_sh.at[pl.ds(rbase + i * ZR, ZR)])
            return carry

        lax.fori_loop(0, rpt // ZR, zc, 0)
        plsc.subcore_barrier()

        ebase = wid * e_per

        def chunk(j, carry):
            cb = ebase + j * C
            pltpu.sync_copy(src_h.at[pl.ds(cb, C)], src_v)
            pltpu.sync_copy(dst_h.at[pl.ds(cb, C)], dst_v)
            pltpu.sync_copy(nrm_h.at[pl.ds(cb, C)], nrm_v)
            pltpu.async_copy(h_h.at[src_v], rows_v, sem).wait()

            def scale(g, carry2):
                nv = nrm_v[pl.ds(g * _L, _L)]
                for e in range(_L):
                    nb = jnp.full((_L,), nv[e], jnp.float32)
                    r = g * _L + e
                    for j in range(_DG):
                        sl = pl.ds(j * _L, _L)
                        rows_v[r, sl] = rows_v[r, sl] * nb
                return carry2

            lax.fori_loop(0, C // _L, scale, 0)
            pltpu.sync_copy(rows_v, acc_sh.at[dst_v], add=True)
            return carry

        lax.fori_loop(0, n_chunks, chunk, 0)
        plsc.subcore_barrier()
        pltpu.sync_copy(acc_sh.at[pl.ds(rbase, rpt)],
                        out_h.at[c, pl.ds(rbase, rpt)])

    return k(h, src, dst, norm)


def _pair_products(h, ps, pd, ns, nd):
    P = ps.shape[0]
    C = 80
    total = P // C
    iters = (total + _NW - 1) // _NW

    @functools.partial(
        pl.kernel,
        compiler_params=pltpu.CompilerParams(needs_layout_passes=False),
        out_type=(jax.ShapeDtypeStruct((P, _D), jnp.float32),
                  jax.ShapeDtypeStruct((P, _D), jnp.float32)),
        mesh=_sc_mesh(),
        scratch_types=[
            pltpu.VMEM((C,), jnp.int32),
            pltpu.VMEM((C,), jnp.int32),
            pltpu.VMEM((C, _D), jnp.float32),
            pltpu.VMEM((C, _D), jnp.float32),
            pltpu.SemaphoreType.DMA,
        ],
    )
    def k(h_h, ps_h, pd_h, ns_h, nd_h, zp_h, zn_h, a_v, b_v, ra_v, rb_v, sem):
        c = lax.axis_index("c")
        s = lax.axis_index("s")
        wid = c * _NS + s

        def do(pa_h, pb_h, z_h):
            def chunk(t, carry):
                ci = wid + t * _NW

                @pl.when(ci < total)
                def _():
                    cb = ci * C
                    pltpu.sync_copy(pa_h.at[pl.ds(cb, C)], a_v)
                    pltpu.sync_copy(pb_h.at[pl.ds(cb, C)], b_v)
                    pltpu.async_copy(h_h.at[a_v], ra_v, sem).wait()
                    pltpu.async_copy(h_h.at[b_v], rb_v, sem).wait()

                    def mul(e, carry2):
                        for j in range(_DG):
                            sl = pl.ds(j * _L, _L)
                            ra_v[e, sl] = ra_v[e, sl] * rb_v[e, sl]
                        return carry2

                    lax.fori_loop(0, C, mul, 0)
                    pltpu.sync_copy(ra_v, z_h.at[pl.ds(cb, C)])

                return carry

            lax.fori_loop(0, iters, chunk, 0)

        do(ps_h, pd_h, zp_h)
        do(ns_h, nd_h, zn_h)

    return k(h, ps, pd, ns, nd)


def _add_body(a_ref, b_ref, o_ref):
    o_ref[...] = a_ref[...] + b_ref[...]


def _combine(a, b):
    N = a.shape[0]
    blk = 640
    return pl.pallas_call(
        _add_body,
        grid=(N // blk,),
        in_specs=[pl.BlockSpec((blk, _D), lambda i: (i, 0))] * 2,
        out_specs=pl.BlockSpec((blk, _D), lambda i: (i, 0)),
        out_shape=jax.ShapeDtypeStruct((N, _D), jnp.float32),
    )(a, b)


def _tag_linear_body(h0, h1, h2, p3a, p3b, w_ref, b_ref, o_ref, *, relu):
    w = w_ref[...]
    acc = (h0[...] @ w[0:128]
           + h1[...] @ w[128:256]
           + h2[...] @ w[256:384]
           + (p3a[...] + p3b[...]) @ w[384:512]
           + b_ref[...])
    o_ref[...] = jnp.maximum(acc, 0.0) if relu else acc


def _tag_linear(h0, h1, h2, p3a, p3b, W, b, relu):
    N = h0.shape[0]
    blk = 640
    return pl.pallas_call(
        functools.partial(_tag_linear_body, relu=relu),
        grid=(N // blk,),
        in_specs=[pl.BlockSpec((blk, _D), lambda i: (i, 0))] * 5
        + [pl.BlockSpec((4 * _D, _D), lambda i: (0, 0)),
           pl.BlockSpec((_D,), lambda i: (0,))],
        out_specs=pl.BlockSpec((blk, _D), lambda i: (i, 0)),
        out_shape=jax.ShapeDtypeStruct((N, _D), jnp.float32),
    )(h0, h1, h2, p3a, p3b, W, b)


def _pred_body(z_ref, p1_ref, pb1_ref, p2_ref, pb2_ref, p3_ref, pb3_ref, o_ref):
    t = z_ref[...] @ p1_ref[...] + pb1_ref[...]
    t = jnp.where(t > 0, t, 0.2 * t)
    t = t @ p2_ref[...] + pb2_ref[...]
    t = jnp.where(t > 0, t, 0.2 * t)
    o_ref[...] = t @ p3_ref[...] + pb3_ref[...]


def _predictor(z, P1, pb1, P2, pb2, P3, pb3):
    B = z.shape[0]
    blk = 400
    return pl.pallas_call(
        _pred_body,
        grid=(B // blk,),
        in_specs=[
            pl.BlockSpec((blk, _D), lambda i: (i, 0)),
            pl.BlockSpec((_D, 64), lambda i: (0, 0)),
            pl.BlockSpec((64,), lambda i: (0,)),
            pl.BlockSpec((64, 32), lambda i: (0, 0)),
            pl.BlockSpec((32,), lambda i: (0,)),
            pl.BlockSpec((32, 1), lambda i: (0, 0)),
            pl.BlockSpec((1,), lambda i: (0,)),
        ],
        out_specs=pl.BlockSpec((blk, 1), lambda i: (i, 0)),
        out_shape=jax.ShapeDtypeStruct((B, 1), jnp.float32),
    )(z, P1, pb1, P2, pb2, P3, pb3)


def kernel(x, edge_index, edge_weight, pos_edges, neg_edges,
           W1, b1, W2, b2, W3, b3, P1, pb1, P2, pb2, P3, pb3):
    N = x.shape[0]
    n_pad = ((N + _NW * _L - 1) // (_NW * _L)) * (_NW * _L)
    src = edge_index[0].astype(jnp.int32)
    dst = edge_index[1].astype(jnp.int32)
    w = edge_weight.astype(jnp.float32)

    parts = _deg_partials(src, dst, w, n_pad)
    norm = _edge_norm(parts, src, dst, w, n_pad)

    h = jnp.pad(x, ((0, n_pad - N), (0, 0)))
    for W, b, act in ((W1, b1, True), (W2, b2, True), (W3, b3, False)):
        f0 = h
        p1 = _prop(f0, src, dst, norm)
        h1 = _combine(p1[0], p1[1])
        p2 = _prop(h1, src, dst, norm)
        h2 = _combine(p2[0], p2[1])
        p3 = _prop(h2, src, dst, norm)
        h = _tag_linear(f0, h1, h2, p3[0], p3[1], W, b, act)

    pe = pos_edges.astype(jnp.int32)
    ne = neg_edges.astype(jnp.int32)
    zp, zn = _pair_products(h, pe[0], pe[1], ne[0], ne[1])
    h_pos = _predictor(zp, P1, pb1, P2, pb2, P3, pb3)
    h_neg = _predictor(zn, P1, pb1, P2, pb2, P3, pb3)
    return (h_pos, h_neg, h[:N])


# speedup vs baseline: 9.8878x; 2.2453x over previous
import functools

import jax
import jax.numpy as jnp
from jax import lax
from jax.experimental import pallas as pl
from jax.experimental.pallas import tpu as pltpu
from jax.experimental.pallas import tpu_sc as plsc

_NC, _NS, _L = 2, 16, 16
_NW = _NC * _NS
_D = 128
_DG = _D // _L


def _sc_mesh():
    return plsc.VectorSubcoreMesh(core_axis_name="c", subcore_axis_name="s")


def _rsqrt16(x):
    xi = plsc.bitcast(x, jnp.int32)
    yi = 0x5F3759DF - lax.shift_right_arithmetic(xi, 1)
    y = plsc.bitcast(yi, jnp.float32)
    for _ in range(3):
        y = y * (1.5 - 0.5 * x * y * y)
    return y


def _deg_partials(src, dst, w, n_pad):
    E = src.shape[0]
    e_per = E // _NW

    @functools.partial(
        pl.kernel,
        compiler_params=pltpu.CompilerParams(needs_layout_passes=False),
        out_type=jax.ShapeDtypeStruct((_NW, 2, n_pad), jnp.float32),
        mesh=_sc_mesh(),
        scratch_types=[
            pltpu.VMEM((e_per,), jnp.int32),
            pltpu.VMEM((e_per,), jnp.int32),
            pltpu.VMEM((e_per,), jnp.float32),
            pltpu.VMEM((n_pad,), jnp.float32),
            pltpu.VMEM((n_pad,), jnp.float32),
        ],
    )
    def k(src_h, dst_h, w_h, out_h, src_v, dst_v, w_v, ds_v, dd_v):
        c = lax.axis_index("c")
        s = lax.axis_index("s")
        wid = c * _NS + s
        base = wid * e_per
        z = jnp.zeros((_L,), jnp.float32)

        def zero(i, carry):
            ds_v[pl.ds(i * _L, _L)] = z
            dd_v[pl.ds(i * _L, _L)] = z
            return carry

        lax.fori_loop(0, n_pad // _L, zero, 0)
        pltpu.sync_copy(src_h.at[pl.ds(base, e_per)], src_v)
        pltpu.sync_copy(dst_h.at[pl.ds(base, e_per)], dst_v)
        pltpu.sync_copy(w_h.at[pl.ds(base, e_per)], w_v)

        def body(i, carry):
            sl = pl.ds(i * _L, _L)
            wv = w_v[sl]
            plsc.addupdate_scatter(ds_v, [src_v[sl]], wv)
            plsc.addupdate_scatter(dd_v, [dst_v[sl]], wv)
            return carry

        lax.fori_loop(0, e_per // _L, body, 0)
        pltpu.sync_copy(ds_v, out_h.at[wid, 0])
        pltpu.sync_copy(dd_v, out_h.at[wid, 1])

    return k(src, dst, w)


def _edge_norm(parts, src, dst, w, n_pad):
    E = src.shape[0]
    e_per = E // _NW
    npc = n_pad // _NS
    CH = 2000

    @functools.partial(
        pl.kernel,
        compiler_params=pltpu.CompilerParams(needs_layout_passes=False),
        out_type=jax.ShapeDtypeStruct((E,), jnp.float32),
        mesh=_sc_mesh(),
        scratch_types=[
            pltpu.VMEM((_NW, 2, npc), jnp.float32),
            pltpu.VMEM((2, npc), jnp.float32),
            pltpu.VMEM((n_pad,), jnp.float32),
            pltpu.VMEM((n_pad,), jnp.float32),
            pltpu.VMEM_SHARED((2, n_pad), jnp.float32),
            pltpu.VMEM((CH,), jnp.int32),
            pltpu.VMEM((CH,), jnp.int32),
            pltpu.VMEM((CH,), jnp.float32),
            pltpu.VMEM((CH,), jnp.float32),
        ],
    )
    def k(parts_h, src_h, dst_h, w_h, norm_h, stage_v, rsl_v, rss_v, rsd_v,
          rs_sh, src_v, dst_v, w_v, nrm_v):
        c = lax.axis_index("c")
        s = lax.axis_index("s")
        wid = c * _NS + s
        nbase = s * npc

        def ldp(p, carry):
            pltpu.sync_copy(parts_h.at[p, 0, pl.ds(nbase, npc)], stage_v.at[p, 0])
            pltpu.sync_copy(parts_h.at[p, 1, pl.ds(nbase, npc)], stage_v.at[p, 1])
            return carry

        lax.fori_loop(0, _NW, ldp, 0)

        def red(i, carry):
            sl = pl.ds(i * _L, _L)

            def acc(p, ab):
                return (ab[0] + stage_v[p, 0, sl], ab[1] + stage_v[p, 1, sl])

            zz = jnp.zeros((_L,), jnp.float32)
            a, b = lax.fori_loop(0, _NW, acc, (zz, zz))
            rsl_v[0, sl] = _rsqrt16(jnp.maximum(a, 1e-6))
            rsl_v[1, sl] = _rsqrt16(jnp.maximum(b, 1e-6))
            return carry

        lax.fori_loop(0, npc // _L, red, 0)
        pltpu.sync_copy(rsl_v.at[0], rs_sh.at[0, pl.ds(nbase, npc)])
        pltpu.sync_copy(rsl_v.at[1], rs_sh.at[1, pl.ds(nbase, npc)])
        plsc.subcore_barrier()
        pltpu.sync_copy(rs_sh.at[0], rss_v)
        pltpu.sync_copy(rs_sh.at[1], rsd_v)

        ebase = wid * e_per

        def chunk(j, carry):
            cb = ebase + j * CH
            pltpu.sync_copy(src_h.at[pl.ds(cb, CH)], src_v)
            pltpu.sync_copy(dst_h.at[pl.ds(cb, CH)], dst_v)
            pltpu.sync_copy(w_h.at[pl.ds(cb, CH)], w_v)

            def inner(i, carry2):
                sl = pl.ds(i * _L, _L)
                a = plsc.load_gather(rss_v, [src_v[sl]])
                b = plsc.load_gather(rsd_v, [dst_v[sl]])
                nrm_v[sl] = w_v[sl] * a * b
                return carry2

            lax.fori_loop(0, CH // _L, inner, 0)
            pltpu.sync_copy(nrm_v, norm_h.at[pl.ds(cb, CH)])
            return carry

        lax.fori_loop(0, e_per // CH, chunk, 0)

    return k(parts, src, dst, w)


def _prop(h, src_flat, dst3, nrm_flat):
    N = h.shape[0]
    _, nch, C = dst3.shape
    e_per = nch * C
    rpt = N // _NS

    @functools.partial(
        pl.kernel,
        compiler_params=pltpu.CompilerParams(needs_layout_passes=False),
        out_type=jax.ShapeDtypeStruct((_NC, N, _D), jnp.float32),
        mesh=_sc_mesh(),
        scratch_types=[
            pltpu.VMEM_SHARED((N, _D), jnp.float32),
            pltpu.VMEM((nch, C), jnp.int32),
            pltpu.VMEM((C,), jnp.int32),
            pltpu.VMEM((C,), jnp.int32),
            pltpu.VMEM((C,), jnp.float32),
            pltpu.VMEM((C,), jnp.float32),
            pltpu.VMEM((C, _D), jnp.float32),
            pltpu.VMEM((C, _D), jnp.float32),
            pltpu.SemaphoreType.DMA,
            pltpu.SemaphoreType.DMA,
            pltpu.SemaphoreType.DMA,
            pltpu.SemaphoreType.DMA,
            pltpu.SemaphoreType.DMA,
            pltpu.SemaphoreType.DMA,
            pltpu.SemaphoreType.DMA,
            pltpu.SemaphoreType.DMA,
            pltpu.SemaphoreType.DMA,
        ],
    )
    def k(h_h, src_h, dst_h, nrm_h, out_h, acc_sh, dst_v, sr0_v, sr1_v,
          nr0_v, nr1_v, r0_v, r1_v, sem_st, sem_g0, sem_g1, sem_s0, sem_s1,
          sem_n0, sem_n1, sem_r0, sem_r1):
        c = lax.axis_index("c")
        s = lax.axis_index("s")
        wid = c * _NS + s
        z = jnp.zeros((_L,), jnp.float32)
        ebase = wid * e_per

        pltpu.async_copy(dst_h.at[wid], dst_v, sem_st)

        def zb(r, carry):
            for j in range(_DG):
                r0_v[r, pl.ds(j * _L, _L)] = z
            return carry

        lax.fori_loop(0, C, zb, 0)
        rbase = s * rpt

        def zc(i, carry):
            pltpu.sync_copy(r0_v, acc_sh.at[pl.ds(rbase + i * C, C)])
            return carry

        lax.fori_loop(0, rpt // C, zc, 0)
        pltpu.make_async_copy(dst_h.at[wid], dst_v, sem_st).wait()
        plsc.subcore_barrier()

        bufs = (r0_v, r1_v)
        srings = (sr0_v, sr1_v)
        nbufs = (nr0_v, nr1_v)
        gsems = (sem_g0, sem_g1)
        ssems = (sem_s0, sem_s1)
        nsems = (sem_n0, sem_n1)
        rsems = (sem_r0, sem_r1)

        def pref(j, b):
            pltpu.async_copy(src_h.at[pl.ds(ebase + j * C, C)], srings[b],
                             rsems[b])
            pltpu.async_copy(nrm_h.at[pl.ds(ebase + j * C, C)], nbufs[b],
                             nsems[b])

        def wait_pref(j, b):
            pltpu.make_async_copy(src_h.at[pl.ds(ebase + j * C, C)],
                                  srings[b], rsems[b]).wait()
            pltpu.make_async_copy(nrm_h.at[pl.ds(ebase + j * C, C)],
                                  nbufs[b], nsems[b]).wait()

        def gather(b):
            pltpu.async_copy(h_h.at[srings[b]], bufs[b], gsems[b])

        def wait_gather(b):
            pltpu.make_async_copy(h_h.at[srings[b]], bufs[b],
                                  gsems[b]).wait()

        def scatter(j, b):
            pltpu.async_copy(bufs[b], acc_sh.at[dst_v.at[j]], ssems[b],
                             add=True)

        def wait_scatter(j, b):
            pltpu.make_async_copy(bufs[b], acc_sh.at[dst_v.at[j]],
                                  ssems[b]).wait()

        def scale(b):
            rows = bufs[b]
            nrm = nbufs[b]

            def grp(g, carry):
                nv = nrm[pl.ds(g * _L, _L)]
                for e in range(_L):
                    nb = jnp.full((_L,), nv[e], jnp.float32)
                    r = g * _L + e
                    for q in range(_DG):
                        sl = pl.ds(q * _L, _L)
                        rows[r, sl] = rows[r, sl] * nb
                return carry

            lax.fori_loop(0, C // _L, grp, 0)

        pref(0, 0)
        pref(1, 1)
        wait_pref(0, 0)
        gather(0)

        def pair(t, carry):
            for b in range(2):
                j = 2 * t + b
                wait_gather(b)

                @pl.when(j >= 1)
                def _():
                    wait_scatter(j - 1, 1 - b)

                @pl.when(j + 2 < nch)
                def _():
                    pref(j + 2, b)

                wait_pref(j + 1, 1 - b)
                gather(1 - b)
                scale(b)
                scatter(j, b)
            return carry

        lax.fori_loop(0, (nch - 1) // 2, pair, 0)
        jt = nch - 1
        wait_gather(0)
        wait_scatter(jt - 1, 1)
        scale(0)
        scatter(jt, 0)
        wait_scatter(jt, 0)
        plsc.subcore_barrier()
        nwb = rpt // C

        def wb_slice(i):
            return pl.ds(rbase + i * C, C)

        pltpu.async_copy(acc_sh.at[wb_slice(0)], bufs[0], gsems[0])
        for i in range(nwb):
            b = i % 2
            pltpu.make_async_copy(acc_sh.at[wb_slice(i)], bufs[b],
                                  gsems[b]).wait()
            if i + 1 < nwb:
                if i >= 1:
                    pltpu.make_async_copy(bufs[1 - b],
                                          out_h.at[c, wb_slice(i - 1)],
                                          ssems[1 - b]).wait()
                pltpu.async_copy(acc_sh.at[wb_slice(i + 1)], bufs[1 - b],
                                 gsems[1 - b])
            pltpu.async_copy(bufs[b], out_h.at[c, wb_slice(i)], ssems[b])
        for i in (nwb - 2, nwb - 1):
            pltpu.make_async_copy(bufs[i % 2], out_h.at[c, wb_slice(i)],
                                  ssems[i % 2]).wait()

    return k(h, src_flat, dst3, nrm_flat)


def _pair_products(h, ps, pd, ns, nd):
    P = ps.shape[0]
    C = 80
    total = P // C
    iters = (total + _NW - 1) // _NW

    @functools.partial(
        pl.kernel,
        compiler_params=pltpu.CompilerParams(needs_layout_passes=False),
        out_type=(jax.ShapeDtypeStruct((P, _D), jnp.float32),
                  jax.ShapeDtypeStruct((P, _D), jnp.float32)),
        mesh=_sc_mesh(),
        scratch_types=[
            pltpu.VMEM((C,), jnp.int32),
            pltpu.VMEM((C,), jnp.int32),
            pltpu.VMEM((C, _D), jnp.float32),
            pltpu.VMEM((C, _D), jnp.float32),
            pltpu.SemaphoreType.DMA,
        ],
    )
    def k(h_h, ps_h, pd_h, ns_h, nd_h, zp_h, zn_h, a_v, b_v, ra_v, rb_v, sem):
        c = lax.axis_index("c")
        s = lax.axis_index("s")
        wid = c * _NS + s

        def do(pa_h, pb_h, z_h):
            def chunk(t, carry):
                ci = wid + t * _NW

                @pl.when(ci < total)
                def _():
                    cb = ci * C
                    pltpu.sync_copy(pa_h.at[pl.ds(cb, C)], a_v)
                    pltpu.sync_copy(pb_h.at[pl.ds(cb, C)], b_v)
                    pltpu.async_copy(h_h.at[a_v], ra_v, sem).wait()
                    pltpu.async_copy(h_h.at[b_v], rb_v, sem).wait()

                    def mul(e, carry2):
                        for j in range(_DG):
                            sl = pl.ds(j * _L, _L)
                            ra_v[e, sl] = ra_v[e, sl] * rb_v[e, sl]
                        return carry2

                    lax.fori_loop(0, C, mul, 0)
                    pltpu.sync_copy(ra_v, z_h.at[pl.ds(cb, C)])

                return carry

            lax.fori_loop(0, iters, chunk, 0)

        do(ps_h, pd_h, zp_h)
        do(ns_h, nd_h, zn_h)

    return k(h, ps, pd, ns, nd)


def _add_body(a_ref, b_ref, o_ref):
    o_ref[...] = a_ref[...] + b_ref[...]


def _combine(a, b):
    N = a.shape[0]
    blk = 640
    return pl.pallas_call(
        _add_body,
        grid=(N // blk,),
        in_specs=[pl.BlockSpec((blk, _D), lambda i: (i, 0))] * 2,
        out_specs=pl.BlockSpec((blk, _D), lambda i: (i, 0)),
        out_shape=jax.ShapeDtypeStruct((N, _D), jnp.float32),
    )(a, b)


def _tag_linear_body(h0, h1, h2, p3a, p3b, w_ref, b_ref, o_ref, *, relu):
    w = w_ref[...]
    acc = (h0[...] @ w[0:128]
           + h1[...] @ w[128:256]
           + h2[...] @ w[256:384]
           + (p3a[...] + p3b[...]) @ w[384:512]
           + b_ref[...])
    o_ref[...] = jnp.maximum(acc, 0.0) if relu else acc


def _tag_linear(h0, h1, h2, p3a, p3b, W, b, relu):
    N = h0.shape[0]
    blk = 640
    return pl.pallas_call(
        functools.partial(_tag_linear_body, relu=relu),
        grid=(N // blk,),
        in_specs=[pl.BlockSpec((blk, _D), lambda i: (i, 0))] * 5
        + [pl.BlockSpec((4 * _D, _D), lambda i: (0, 0)),
           pl.BlockSpec((_D,), lambda i: (0,))],
        out_specs=pl.BlockSpec((blk, _D), lambda i: (i, 0)),
        out_shape=jax.ShapeDtypeStruct((N, _D), jnp.float32),
    )(h0, h1, h2, p3a, p3b, W, b)


def _pred_body(z_ref, p1_ref, pb1_ref, p2_ref, pb2_ref, p3_ref, pb3_ref, o_ref):
    t = z_ref[...] @ p1_ref[...] + pb1_ref[...]
    t = jnp.where(t > 0, t, 0.2 * t)
    t = t @ p2_ref[...] + pb2_ref[...]
    t = jnp.where(t > 0, t, 0.2 * t)
    o_ref[...] = t @ p3_ref[...] + pb3_ref[...]


def _predictor(z, P1, pb1, P2, pb2, P3, pb3):
    B = z.shape[0]
    blk = 400
    return pl.pallas_call(
        _pred_body,
        grid=(B // blk,),
        in_specs=[
            pl.BlockSpec((blk, _D), lambda i: (i, 0)),
            pl.BlockSpec((_D, 64), lambda i: (0, 0)),
            pl.BlockSpec((64,), lambda i: (0,)),
            pl.BlockSpec((64, 32), lambda i: (0, 0)),
            pl.BlockSpec((32,), lambda i: (0,)),
            pl.BlockSpec((32, 1), lambda i: (0, 0)),
            pl.BlockSpec((1,), lambda i: (0,)),
        ],
        out_specs=pl.BlockSpec((blk, 1), lambda i: (i, 0)),
        out_shape=jax.ShapeDtypeStruct((B, 1), jnp.float32),
    )(z, P1, pb1, P2, pb2, P3, pb3)


def kernel(x, edge_index, edge_weight, pos_edges, neg_edges,
           W1, b1, W2, b2, W3, b3, P1, pb1, P2, pb2, P3, pb3):
    N = x.shape[0]
    n_pad = ((N + _NW * _L - 1) // (_NW * _L)) * (_NW * _L)
    src = edge_index[0].astype(jnp.int32)
    dst = edge_index[1].astype(jnp.int32)
    w = edge_weight.astype(jnp.float32)

    parts = _deg_partials(src, dst, w, n_pad)
    norm = _edge_norm(parts, src, dst, w, n_pad)

    E = src.shape[0]
    C = 80
    nch = E // (_NW * C)
    dst3 = dst.reshape(_NW, nch, C)

    h = jnp.pad(x, ((0, n_pad - N), (0, 0)))
    for W, b, act in ((W1, b1, True), (W2, b2, True), (W3, b3, False)):
        f0 = h
        p1 = _prop(f0, src, dst3, norm)
        h1 = _combine(p1[0], p1[1])
        p2 = _prop(h1, src, dst3, norm)
        h2 = _combine(p2[0], p2[1])
        p3 = _prop(h2, src, dst3, norm)
        h = _tag_linear(f0, h1, h2, p3[0], p3[1], W, b, act)

    pe = pos_edges.astype(jnp.int32)
    ne = neg_edges.astype(jnp.int32)
    zp, zn = _pair_products(h, pe[0], pe[1], ne[0], ne[1])
    h_pos = _predictor(zp, P1, pb1, P2, pb2, P3, pb3)
    h_neg = _predictor(zn, P1, pb1, P2, pb2, P3, pb3)
    return (h_pos, h_neg, h[:N])


# speedup vs baseline: 10.7540x; 1.0876x over previous
import functools

import jax
import jax.numpy as jnp
from jax import lax
from jax.experimental import pallas as pl
from jax.experimental.pallas import tpu as pltpu
from jax.experimental.pallas import tpu_sc as plsc

_NC, _NS, _L = 2, 16, 16
_NW = _NC * _NS
_D = 128
_DG = _D // _L


def _sc_mesh():
    return plsc.VectorSubcoreMesh(core_axis_name="c", subcore_axis_name="s")


def _rsqrt16(x):
    xi = plsc.bitcast(x, jnp.int32)
    yi = 0x5F3759DF - lax.shift_right_arithmetic(xi, 1)
    y = plsc.bitcast(yi, jnp.float32)
    for _ in range(3):
        y = y * (1.5 - 0.5 * x * y * y)
    return y


def _deg_partials(src, dst, w, n_pad):
    E = src.shape[0]
    e_per = E // _NW

    @functools.partial(
        pl.kernel,
        compiler_params=pltpu.CompilerParams(needs_layout_passes=False),
        out_type=jax.ShapeDtypeStruct((_NW, 2, n_pad), jnp.float32),
        mesh=_sc_mesh(),
        scratch_types=[
            pltpu.VMEM((e_per,), jnp.int32),
            pltpu.VMEM((e_per,), jnp.int32),
            pltpu.VMEM((e_per,), jnp.float32),
            pltpu.VMEM((n_pad,), jnp.float32),
            pltpu.VMEM((n_pad,), jnp.float32),
        ],
    )
    def k(src_h, dst_h, w_h, out_h, src_v, dst_v, w_v, ds_v, dd_v):
        c = lax.axis_index("c")
        s = lax.axis_index("s")
        wid = c * _NS + s
        base = wid * e_per
        z = jnp.zeros((_L,), jnp.float32)

        def zero(i, carry):
            ds_v[pl.ds(i * _L, _L)] = z
            dd_v[pl.ds(i * _L, _L)] = z
            return carry

        lax.fori_loop(0, n_pad // _L, zero, 0)
        pltpu.sync_copy(src_h.at[pl.ds(base, e_per)], src_v)
        pltpu.sync_copy(dst_h.at[pl.ds(base, e_per)], dst_v)
        pltpu.sync_copy(w_h.at[pl.ds(base, e_per)], w_v)

        def body(i, carry):
            sl = pl.ds(i * _L, _L)
            wv = w_v[sl]
            plsc.addupdate_scatter(ds_v, [src_v[sl]], wv)
            plsc.addupdate_scatter(dd_v, [dst_v[sl]], wv)
            return carry

        lax.fori_loop(0, e_per // _L, body, 0)
        pltpu.sync_copy(ds_v, out_h.at[wid, 0])
        pltpu.sync_copy(dd_v, out_h.at[wid, 1])

    return k(src, dst, w)


def _edge_norm(parts, src, dst, w, n_pad):
    E = src.shape[0]
    e_per = E // _NW
    npc = n_pad // _NS
    CH = 2000

    @functools.partial(
        pl.kernel,
        compiler_params=pltpu.CompilerParams(needs_layout_passes=False),
        out_type=jax.ShapeDtypeStruct((E,), jnp.float32),
        mesh=_sc_mesh(),
        scratch_types=[
            pltpu.VMEM((_NW, 2, npc), jnp.float32),
            pltpu.VMEM((2, npc), jnp.float32),
            pltpu.VMEM((n_pad,), jnp.float32),
            pltpu.VMEM((n_pad,), jnp.float32),
            pltpu.VMEM_SHARED((2, n_pad), jnp.float32),
            pltpu.VMEM((CH,), jnp.int32),
            pltpu.VMEM((CH,), jnp.int32),
            pltpu.VMEM((CH,), jnp.float32),
            pltpu.VMEM((CH,), jnp.float32),
        ],
    )
    def k(parts_h, src_h, dst_h, w_h, norm_h, stage_v, rsl_v, rss_v, rsd_v,
          rs_sh, src_v, dst_v, w_v, nrm_v):
        c = lax.axis_index("c")
        s = lax.axis_index("s")
        wid = c * _NS + s
        nbase = s * npc

        def ldp(p, carry):
            pltpu.sync_copy(parts_h.at[p, 0, pl.ds(nbase, npc)], stage_v.at[p, 0])
            pltpu.sync_copy(parts_h.at[p, 1, pl.ds(nbase, npc)], stage_v.at[p, 1])
            return carry

        lax.fori_loop(0, _NW, ldp, 0)

        def red(i, carry):
            sl = pl.ds(i * _L, _L)

            def acc(p, ab):
                return (ab[0] + stage_v[p, 0, sl], ab[1] + stage_v[p, 1, sl])

            zz = jnp.zeros((_L,), jnp.float32)
            a, b = lax.fori_loop(0, _NW, acc, (zz, zz))
            rsl_v[0, sl] = _rsqrt16(jnp.maximum(a, 1e-6))
            rsl_v[1, sl] = _rsqrt16(jnp.maximum(b, 1e-6))
            return carry

        lax.fori_loop(0, npc // _L, red, 0)
        pltpu.sync_copy(rsl_v.at[0], rs_sh.at[0, pl.ds(nbase, npc)])
        pltpu.sync_copy(rsl_v.at[1], rs_sh.at[1, pl.ds(nbase, npc)])
        plsc.subcore_barrier()
        pltpu.sync_copy(rs_sh.at[0], rss_v)
        pltpu.sync_copy(rs_sh.at[1], rsd_v)

        ebase = wid * e_per

        def chunk(j, carry):
            cb = ebase + j * CH
            pltpu.sync_copy(src_h.at[pl.ds(cb, CH)], src_v)
            pltpu.sync_copy(dst_h.at[pl.ds(cb, CH)], dst_v)
            pltpu.sync_copy(w_h.at[pl.ds(cb, CH)], w_v)

            def inner(i, carry2):
                sl = pl.ds(i * _L, _L)
                a = plsc.load_gather(rss_v, [src_v[sl]])
                b = plsc.load_gather(rsd_v, [dst_v[sl]])
                nrm_v[sl] = w_v[sl] * a * b
                return carry2

            lax.fori_loop(0, CH // _L, inner, 0)
            pltpu.sync_copy(nrm_v, norm_h.at[pl.ds(cb, CH)])
            return carry

        lax.fori_loop(0, e_per // CH, chunk, 0)

    return k(parts, src, dst, w)


def _prop(h, src_flat, dst3, nrm_flat):
    N = h.shape[0]
    _, nch, C = dst3.shape
    e_per = nch * C
    rpt = N // _NS

    @functools.partial(
        pl.kernel,
        compiler_params=pltpu.CompilerParams(needs_layout_passes=False),
        out_type=jax.ShapeDtypeStruct((_NC, N, _D), jnp.float32),
        mesh=_sc_mesh(),
        scratch_types=[
            pltpu.VMEM_SHARED((N, _D), jnp.float32),
            pltpu.VMEM((nch, C), jnp.int32),
            pltpu.VMEM((C,), jnp.int32),
            pltpu.VMEM((C,), jnp.int32),
            pltpu.VMEM((C,), jnp.float32),
            pltpu.VMEM((C,), jnp.float32),
            pltpu.VMEM((C, _D), jnp.float32),
            pltpu.VMEM((C, _D), jnp.float32),
            pltpu.SemaphoreType.DMA,
            pltpu.SemaphoreType.DMA,
            pltpu.SemaphoreType.DMA,
            pltpu.SemaphoreType.DMA,
            pltpu.SemaphoreType.DMA,
            pltpu.SemaphoreType.DMA,
            pltpu.SemaphoreType.DMA,
            pltpu.SemaphoreType.DMA,
            pltpu.SemaphoreType.DMA,
        ],
    )
    def k(h_h, src_h, dst_h, nrm_h, out_h, acc_sh, dst_v, sr0_v, sr1_v,
          nr0_v, nr1_v, r0_v, r1_v, sem_st, sem_g0, sem_g1, sem_s0, sem_s1,
          sem_n0, sem_n1, sem_r0, sem_r1):
        c = lax.axis_index("c")
        s = lax.axis_index("s")
        wid = c * _NS + s
        z = jnp.zeros((_L,), jnp.float32)
        ebase = wid * e_per

        pltpu.async_copy(dst_h.at[wid], dst_v, sem_st)

        def zb(r, carry):
            for j in range(_DG):
                r0_v[r, pl.ds(j * _L, _L)] = z
            return carry

        lax.fori_loop(0, C, zb, 0)
        rbase = s * rpt

        def zc(i, carry):
            pltpu.sync_copy(r0_v, acc_sh.at[pl.ds(rbase + i * C, C)])
            return carry

        lax.fori_loop(0, rpt // C, zc, 0)
        pltpu.make_async_copy(dst_h.at[wid], dst_v, sem_st).wait()
        plsc.subcore_barrier()

        bufs = (r0_v, r1_v)
        srings = (sr0_v, sr1_v)
        nbufs = (nr0_v, nr1_v)
        gsems = (sem_g0, sem_g1)
        ssems = (sem_s0, sem_s1)
        nsems = (sem_n0, sem_n1)
        rsems = (sem_r0, sem_r1)

        def pref(j, b):
            pltpu.async_copy(src_h.at[pl.ds(ebase + j * C, C)], srings[b],
                             rsems[b])
            pltpu.async_copy(nrm_h.at[pl.ds(ebase + j * C, C)], nbufs[b],
                             nsems[b])

        def wait_pref(j, b):
            pltpu.make_async_copy(src_h.at[pl.ds(ebase + j * C, C)],
                                  srings[b], rsems[b]).wait()
            pltpu.make_async_copy(nrm_h.at[pl.ds(ebase + j * C, C)],
                                  nbufs[b], nsems[b]).wait()

        def gather(b):
            pltpu.async_copy(h_h.at[srings[b]], bufs[b], gsems[b])

        def wait_gather(b):
            pltpu.make_async_copy(h_h.at[srings[b]], bufs[b],
                                  gsems[b]).wait()

        def scatter(j, b):
            pltpu.async_copy(bufs[b], acc_sh.at[dst_v.at[j]], ssems[b],
                             add=True)

        def wait_scatter(j, b):
            pltpu.make_async_copy(bufs[b], acc_sh.at[dst_v.at[j]],
                                  ssems[b]).wait()

        def scale(b):
            rows = bufs[b]
            nrm = nbufs[b]

            def grp(g, carry):
                nv = nrm[pl.ds(g * _L, _L)]
                for e in range(_L):
                    nb = jnp.full((_L,), nv[e], jnp.float32)
                    r = g * _L + e
                    for q in range(_DG):
                        sl = pl.ds(q * _L, _L)
                        rows[r, sl] = rows[r, sl] * nb
                return carry

            lax.fori_loop(0, C // _L, grp, 0)

        pref(0, 0)
        pref(1, 1)
        wait_pref(0, 0)
        gather(0)

        def pair(t, carry):
            for b in range(2):
                j = 2 * t + b
                wait_gather(b)

                @pl.when(j >= 1)
                def _():
                    wait_scatter(j - 1, 1 - b)

                @pl.when(j + 2 < nch)
                def _():
                    pref(j + 2, b)

                wait_pref(j + 1, 1 - b)
                gather(1 - b)
                scale(b)
                scatter(j, b)
            return carry

        lax.fori_loop(0, (nch - 1) // 2, pair, 0)
        jt = nch - 1
        wait_gather(0)
        wait_scatter(jt - 1, 1)
        scale(0)
        scatter(jt, 0)
        wait_scatter(jt, 0)
        plsc.subcore_barrier()
        nwb = rpt // C

        def wb_slice(i):
            return pl.ds(rbase + i * C, C)

        pltpu.async_copy(acc_sh.at[wb_slice(0)], bufs[0], gsems[0])
        for i in range(nwb):
            b = i % 2
            pltpu.make_async_copy(acc_sh.at[wb_slice(i)], bufs[b],
                                  gsems[b]).wait()
            if i + 1 < nwb:
                if i >= 1:
                    pltpu.make_async_copy(bufs[1 - b],
                                          out_h.at[c, wb_slice(i - 1)],
                                          ssems[1 - b]).wait()
                pltpu.async_copy(acc_sh.at[wb_slice(i + 1)], bufs[1 - b],
                                 gsems[1 - b])
            pltpu.async_copy(bufs[b], out_h.at[c, wb_slice(i)], ssems[b])
        for i in (nwb - 2, nwb - 1):
            pltpu.make_async_copy(bufs[i % 2], out_h.at[c, wb_slice(i)],
                                  ssems[i % 2]).wait()

    return k(h, src_flat, dst3, nrm_flat)


def _pair_products(h, ps, pd, ns, nd):
    P = ps.shape[0]
    C = 80
    total = P // C
    iters = (total + _NW - 1) // _NW

    @functools.partial(
        pl.kernel,
        compiler_params=pltpu.CompilerParams(needs_layout_passes=False),
        out_type=jax.ShapeDtypeStruct((2 * P, _D), jnp.float32),
        mesh=_sc_mesh(),
        scratch_types=[
            pltpu.VMEM((C,), jnp.int32),
            pltpu.VMEM((C,), jnp.int32),
            pltpu.VMEM((C, _D), jnp.float32),
            pltpu.VMEM((C, _D), jnp.float32),
            pltpu.SemaphoreType.DMA,
        ],
    )
    def k(h_h, ps_h, pd_h, ns_h, nd_h, z_h, a_v, b_v, ra_v, rb_v, sem):
        c = lax.axis_index("c")
        s = lax.axis_index("s")
        wid = c * _NS + s

        def do(pa_h, pb_h, obase):
            def chunk(t, carry):
                ci = wid + t * _NW

                @pl.when(ci < total)
                def _():
                    cb = ci * C
                    pltpu.sync_copy(pa_h.at[pl.ds(cb, C)], a_v)
                    pltpu.sync_copy(pb_h.at[pl.ds(cb, C)], b_v)
                    pltpu.async_copy(h_h.at[a_v], ra_v, sem).wait()
                    pltpu.async_copy(h_h.at[b_v], rb_v, sem).wait()

                    def mul(e, carry2):
                        for j in range(_DG):
                            sl = pl.ds(j * _L, _L)
                            ra_v[e, sl] = ra_v[e, sl] * rb_v[e, sl]
                        return carry2

                    lax.fori_loop(0, C, mul, 0)
                    pltpu.sync_copy(ra_v, z_h.at[pl.ds(obase + cb, C)])

                return carry

            lax.fori_loop(0, iters, chunk, 0)

        do(ps_h, pd_h, 0)
        do(ns_h, nd_h, P)

    return k(h, ps, pd, ns, nd)


def _add_body(a_ref, b_ref, o_ref):
    o_ref[...] = a_ref[0] + b_ref[0]


def _combine(p):
    N = p.shape[1]
    blk = 2048
    return pl.pallas_call(
        _add_body,
        grid=(N // blk,),
        in_specs=[pl.BlockSpec((1, blk, _D), lambda i: (0, i, 0)),
                  pl.BlockSpec((1, blk, _D), lambda i: (1, i, 0))],
        out_specs=pl.BlockSpec((blk, _D), lambda i: (i, 0)),
        out_shape=jax.ShapeDtypeStruct((N, _D), jnp.float32),
    )(p, p)


def _tag_linear_body(h0, h1, h2, p3a, p3b, w_ref, b_ref, o_ref, *, relu):
    w = w_ref[...]
    acc = (h0[...] @ w[0:128]
           + h1[...] @ w[128:256]
           + h2[...] @ w[256:384]
           + (p3a[0] + p3b[0]) @ w[384:512]
           + b_ref[...])
    o_ref[...] = jnp.maximum(acc, 0.0) if relu else acc


def _tag_linear(h0, h1, h2, p3, W, b, relu):
    N = h0.shape[0]
    blk = 2048
    return pl.pallas_call(
        functools.partial(_tag_linear_body, relu=relu),
        grid=(N // blk,),
        in_specs=[pl.BlockSpec((blk, _D), lambda i: (i, 0))] * 3
        + [pl.BlockSpec((1, blk, _D), lambda i: (0, i, 0)),
           pl.BlockSpec((1, blk, _D), lambda i: (1, i, 0)),
           pl.BlockSpec((4 * _D, _D), lambda i: (0, 0)),
           pl.BlockSpec((_D,), lambda i: (0,))],
        out_specs=pl.BlockSpec((blk, _D), lambda i: (i, 0)),
        out_shape=jax.ShapeDtypeStruct((N, _D), jnp.float32),
    )(h0, h1, h2, p3, p3, W, b)


def _pred_body(z_ref, p1_ref, pb1_ref, p2_ref, pb2_ref, p3_ref, pb3_ref, o_ref):
    t = z_ref[...] @ p1_ref[...] + pb1_ref[...]
    t = jnp.where(t > 0, t, 0.2 * t)
    t = t @ p2_ref[...] + pb2_ref[...]
    t = jnp.where(t > 0, t, 0.2 * t)
    o_ref[...] = t @ p3_ref[...] + pb3_ref[...]


def _predictor(z, P1, pb1, P2, pb2, P3, pb3):
    B = z.shape[0]
    blk = 2000
    return pl.pallas_call(
        _pred_body,
        grid=(B // blk,),
        in_specs=[
            pl.BlockSpec((blk, _D), lambda i: (i, 0)),
            pl.BlockSpec((_D, 64), lambda i: (0, 0)),
            pl.BlockSpec((64,), lambda i: (0,)),
            pl.BlockSpec((64, 32), lambda i: (0, 0)),
            pl.BlockSpec((32,), lambda i: (0,)),
            pl.BlockSpec((32, 1), lambda i: (0, 0)),
            pl.BlockSpec((1,), lambda i: (0,)),
        ],
        out_specs=pl.BlockSpec((blk, 1), lambda i: (i, 0)),
        out_shape=jax.ShapeDtypeStruct((B, 1), jnp.float32),
    )(z, P1, pb1, P2, pb2, P3, pb3)


def kernel(x, edge_index, edge_weight, pos_edges, neg_edges,
           W1, b1, W2, b2, W3, b3, P1, pb1, P2, pb2, P3, pb3):
    N = x.shape[0]
    n_pad = ((N + _NW * _L - 1) // (_NW * _L)) * (_NW * _L)
    src = edge_index[0].astype(jnp.int32)
    dst = edge_index[1].astype(jnp.int32)
    w = edge_weight.astype(jnp.float32)

    parts = _deg_partials(src, dst, w, n_pad)
    norm = _edge_norm(parts, src, dst, w, n_pad)

    E = src.shape[0]
    C = 80
    nch = E // (_NW * C)
    dst3 = dst.reshape(_NW, nch, C)

    h = jnp.pad(x, ((0, n_pad - N), (0, 0)))
    for W, b, act in ((W1, b1, True), (W2, b2, True), (W3, b3, False)):
        f0 = h
        p1 = _prop(f0, src, dst3, norm)
        h1 = _combine(p1)
        p2 = _prop(h1, src, dst3, norm)
        h2 = _combine(p2)
        p3 = _prop(h2, src, dst3, norm)
        h = _tag_linear(f0, h1, h2, p3, W, b, act)

    pe = pos_edges.astype(jnp.int32)
    ne = neg_edges.astype(jnp.int32)
    z = _pair_products(h, pe[0], pe[1], ne[0], ne[1])
    hz = _predictor(z, P1, pb1, P2, pb2, P3, pb3)
    P = pe.shape[1]
    return (hz[:P], hz[P:], h[:N])


# speedup vs baseline: 12.6343x; 1.1749x over previous
import functools

import jax
import jax.numpy as jnp
from jax import lax
from jax.experimental import pallas as pl
from jax.experimental.pallas import tpu as pltpu
from jax.experimental.pallas import tpu_sc as plsc

_NC, _NS, _L = 2, 16, 16
_NW = _NC * _NS
_D = 128
_DG = _D // _L


def _sc_mesh():
    return plsc.VectorSubcoreMesh(core_axis_name="c", subcore_axis_name="s")


def _rsqrt16(x):
    xi = plsc.bitcast(x, jnp.int32)
    yi = 0x5F3759DF - lax.shift_right_arithmetic(xi, 1)
    y = plsc.bitcast(yi, jnp.float32)
    for _ in range(3):
        y = y * (1.5 - 0.5 * x * y * y)
    return y


def _deg_partials(src, dst, w, n_pad):
    E = src.shape[0]
    e_per = E // _NW

    @functools.partial(
        pl.kernel,
        compiler_params=pltpu.CompilerParams(needs_layout_passes=False),
        out_type=jax.ShapeDtypeStruct((_NW, 2, n_pad), jnp.float32),
        mesh=_sc_mesh(),
        scratch_types=[
            pltpu.VMEM((e_per,), jnp.int32),
            pltpu.VMEM((e_per,), jnp.int32),
            pltpu.VMEM((e_per,), jnp.float32),
            pltpu.VMEM((n_pad,), jnp.float32),
            pltpu.VMEM((n_pad,), jnp.float32),
        ],
    )
    def k(src_h, dst_h, w_h, out_h, src_v, dst_v, w_v, ds_v, dd_v):
        c = lax.axis_index("c")
        s = lax.axis_index("s")
        wid = c * _NS + s
        base = wid * e_per
        z = jnp.zeros((_L,), jnp.float32)

        def zero(i, carry):
            ds_v[pl.ds(i * _L, _L)] = z
            dd_v[pl.ds(i * _L, _L)] = z
            return carry

        lax.fori_loop(0, n_pad // _L, zero, 0)
        pltpu.sync_copy(src_h.at[pl.ds(base, e_per)], src_v)
        pltpu.sync_copy(dst_h.at[pl.ds(base, e_per)], dst_v)
        pltpu.sync_copy(w_h.at[pl.ds(base, e_per)], w_v)

        def body(i, carry):
            sl = pl.ds(i * _L, _L)
            wv = w_v[sl]
            plsc.addupdate_scatter(ds_v, [src_v[sl]], wv)
            plsc.addupdate_scatter(dd_v, [dst_v[sl]], wv)
            return carry

        lax.fori_loop(0, e_per // _L, body, 0)
        pltpu.sync_copy(ds_v, out_h.at[wid, 0])
        pltpu.sync_copy(dd_v, out_h.at[wid, 1])

    return k(src, dst, w)


def _edge_norm(parts, src, dst, w, n_pad):
    E = src.shape[0]
    e_per = E // _NW
    npc = n_pad // _NS
    CH = 2000

    @functools.partial(
        pl.kernel,
        compiler_params=pltpu.CompilerParams(needs_layout_passes=False),
        out_type=jax.ShapeDtypeStruct((E,), jnp.float32),
        mesh=_sc_mesh(),
        scratch_types=[
            pltpu.VMEM((_NW, 2, npc), jnp.float32),
            pltpu.VMEM((2, npc), jnp.float32),
            pltpu.VMEM((n_pad,), jnp.float32),
            pltpu.VMEM((n_pad,), jnp.float32),
            pltpu.VMEM_SHARED((2, n_pad), jnp.float32),
            pltpu.VMEM((CH,), jnp.int32),
            pltpu.VMEM((CH,), jnp.int32),
            pltpu.VMEM((CH,), jnp.float32),
            pltpu.VMEM((CH,), jnp.float32),
        ],
    )
    def k(parts_h, src_h, dst_h, w_h, norm_h, stage_v, rsl_v, rss_v, rsd_v,
          rs_sh, src_v, dst_v, w_v, nrm_v):
        c = lax.axis_index("c")
        s = lax.axis_index("s")
        wid = c * _NS + s
        nbase = s * npc

        def ldp(p, carry):
            pltpu.sync_copy(parts_h.at[p, 0, pl.ds(nbase, npc)], stage_v.at[p, 0])
            pltpu.sync_copy(parts_h.at[p, 1, pl.ds(nbase, npc)], stage_v.at[p, 1])
            return carry

        lax.fori_loop(0, _NW, ldp, 0)

        def red(i, carry):
            sl = pl.ds(i * _L, _L)

            def acc(p, ab):
                return (ab[0] + stage_v[p, 0, sl], ab[1] + stage_v[p, 1, sl])

            zz = jnp.zeros((_L,), jnp.float32)
            a, b = lax.fori_loop(0, _NW, acc, (zz, zz))
            rsl_v[0, sl] = _rsqrt16(jnp.maximum(a, 1e-6))
            rsl_v[1, sl] = _rsqrt16(jnp.maximum(b, 1e-6))
            return carry

        lax.fori_loop(0, npc // _L, red, 0)
        pltpu.sync_copy(rsl_v.at[0], rs_sh.at[0, pl.ds(nbase, npc)])
        pltpu.sync_copy(rsl_v.at[1], rs_sh.at[1, pl.ds(nbase, npc)])
        plsc.subcore_barrier()
        pltpu.sync_copy(rs_sh.at[0], rss_v)
        pltpu.sync_copy(rs_sh.at[1], rsd_v)

        ebase = wid * e_per

        def chunk(j, carry):
            cb = ebase + j * CH
            pltpu.sync_copy(src_h.at[pl.ds(cb, CH)], src_v)
            pltpu.sync_copy(dst_h.at[pl.ds(cb, CH)], dst_v)
            pltpu.sync_copy(w_h.at[pl.ds(cb, CH)], w_v)

            def inner(i, carry2):
                sl = pl.ds(i * _L, _L)
                a = plsc.load_gather(rss_v, [src_v[sl]])
                b = plsc.load_gather(rsd_v, [dst_v[sl]])
                nrm_v[sl] = w_v[sl] * a * b
                return carry2

            lax.fori_loop(0, CH // _L, inner, 0)
            pltpu.sync_copy(nrm_v, norm_h.at[pl.ds(cb, CH)])
            return carry

        lax.fori_loop(0, e_per // CH, chunk, 0)

    return k(parts, src, dst, w)


def _prop(h, src_flat, dst_flat, nrm_flat):
    N = h.shape[0]
    E = src_flat.shape[0]
    e_per = E // _NW
    C = 80
    nch = e_per // C
    rpt = N // _NS

    @functools.partial(
        pl.kernel,
        compiler_params=pltpu.CompilerParams(needs_layout_passes=False),
        out_type=jax.ShapeDtypeStruct((_NC, N, _D), jnp.float32),
        mesh=_sc_mesh(),
        scratch_types=[
            pltpu.VMEM_SHARED((N, _D), jnp.float32),
            pltpu.VMEM((3, C), jnp.int32),
            pltpu.VMEM((3, C), jnp.int32),
            pltpu.VMEM((3, C), jnp.float32),
            pltpu.VMEM((C, _D), jnp.float32),
            pltpu.VMEM((C, _D), jnp.float32),
            pltpu.VMEM((C, _D), jnp.float32),
            [pltpu.SemaphoreType.DMA] * 12,
        ],
    )
    def k(h_h, src_h, dst_h, nrm_h, out_h, acc_sh, sr_v, dr_v, nr_v,
          r0_v, r1_v, r2_v, sems):
        (sem_g0, sem_g1, sem_g2, sem_s0, sem_s1, sem_s2,
         sem_p0, sem_p1, sem_p2, sem_d0, sem_d1, sem_d2) = sems
        c = lax.axis_index("c")
        s = lax.axis_index("s")
        wid = c * _NS + s
        z = jnp.zeros((_L,), jnp.float32)
        ebase = wid * e_per

        def zb(r, carry):
            for j in range(_DG):
                r0_v[r, pl.ds(j * _L, _L)] = z
            return carry

        lax.fori_loop(0, C, zb, 0)
        rbase = s * rpt

        def zc(i, carry):
            pltpu.sync_copy(r0_v, acc_sh.at[pl.ds(rbase + i * C, C)])
            return carry

        lax.fori_loop(0, rpt // C, zc, 0)
        plsc.subcore_barrier()

        bufs = (r0_v, r1_v, r2_v)
        gsems = (sem_g0, sem_g1, sem_g2)
        ssems = (sem_s0, sem_s1, sem_s2)
        psems = (sem_p0, sem_p1, sem_p2)
        dsems = (sem_d0, sem_d1, sem_d2)

        def esl(j):
            return pl.ds(ebase + j * C, C)

        def spref(j, m):
            pltpu.async_copy(src_h.at[esl(j)], sr_v.at[m], psems[m])
            pltpu.async_copy(nrm_h.at[esl(j)], nr_v.at[m], psems[m])

        def wait_spref(j, m):
            pltpu.make_async_copy(src_h.at[esl(j)], sr_v.at[m],
                                  psems[m]).wait()
            pltpu.make_async_copy(nrm_h.at[esl(j)], nr_v.at[m],
                                  psems[m]).wait()

        def dpref(j, m):
            pltpu.async_copy(dst_h.at[esl(j)], dr_v.at[m], dsems[m])

        def wait_dpref(j, m):
            pltpu.make_async_copy(dst_h.at[esl(j)], dr_v.at[m],
                                  dsems[m]).wait()

        def gather(m):
            pltpu.async_copy(h_h.at[sr_v.at[m]], bufs[m], gsems[m])

        def wait_gather(m):
            pltpu.make_async_copy(h_h.at[sr_v.at[m]], bufs[m],
                                  gsems[m]).wait()

        def scatter(m):
            pltpu.async_copy(bufs[m], acc_sh.at[dr_v.at[m]], ssems[m],
                             add=True)

        def wait_scatter(m):
            pltpu.make_async_copy(bufs[m], acc_sh.at[dr_v.at[m]],
                                  ssems[m]).wait()

        def scale(m):
            rows = bufs[m]

            def grp(g, carry):
                nv = nr_v[m, pl.ds(g * _L, _L)]
                for e in range(_L):
                    nb = jnp.full((_L,), nv[e], jnp.float32)
                    r = g * _L + e
                    for q in range(_DG):
                        sl = pl.ds(q * _L, _L)
                        rows[r, sl] = rows[r, sl] * nb
                return carry

            lax.fori_loop(0, C // _L, grp, 0)

        def body(j, b):
            if j + 1 < nch:
                wait_spref(j + 1, (b + 1) % 3)
                gather((b + 1) % 3)
            wait_gather(b)
            scale(b)
            if j >= 1:
                wait_scatter((b + 2) % 3)
            wait_dpref(j, b)
            scatter(b)
            if j + 3 < nch:
                spref(j + 3, b)
            if j + 2 < nch:
                dpref(j + 2, (b + 2) % 3)

        spref(0, 0)
        spref(1, 1)
        spref(2, 2)
        dpref(0, 0)
        dpref(1, 1)
        wait_spref(0, 0)
        gather(0)

        def triple(t, carry):
            j0 = 3 * t
            for b in range(3):
                j = j0 + b

                def wrapped():
                    if b == 0:
                        @pl.when(j >= 1)
                        def _():
                            wait_scatter(2)

                        wait_spref(j + 1, 1)
                        gather(1)
                        wait_gather(0)
                        scale(0)
                        wait_dpref(j, 0)
                        scatter(0)

                        @pl.when(j + 3 < nch)
                        def _():
                            spref(j + 3, 0)

                        dpref(j + 2, 2)
                    else:
                        wait_spref(j + 1, (b + 1) % 3)
                        gather((b + 1) % 3)
                        wait_gather(b)
                        scale(b)
                        wait_scatter((b + 2) % 3)
                        wait_dpref(j, b)
                        scatter(b)

                        @pl.when(j + 3 < nch)
                        def _():
                            spref(j + 3, b)

                        @pl.when(j + 2 < nch)
                        def _():
                            dpref(j + 2, (b + 2) % 3)

                wrapped()
            return carry

        nfull = (nch - 2) // 3
        lax.fori_loop(0, nfull, triple, 0)
        for j in range(3 * nfull, nch):
            body(j, j % 3)
        wait_scatter((nch - 1) % 3)
        plsc.subcore_barrier()
        nwb = rpt // C

        def wb_slice(i):
            return pl.ds(rbase + i * C, C)

        pltpu.async_copy(acc_sh.at[wb_slice(0)], bufs[0], gsems[0])
        for i in range(nwb):
            b = i % 2
            pltpu.make_async_copy(acc_sh.at[wb_slice(i)], bufs[b],
                                  gsems[b]).wait()
            if i + 1 < nwb:
                if i >= 1:
                    pltpu.make_async_copy(bufs[1 - b],
                                          out_h.at[c, wb_slice(i - 1)],
                                          ssems[1 - b]).wait()
                pltpu.async_copy(acc_sh.at[wb_slice(i + 1)], bufs[1 - b],
                                 gsems[1 - b])
            pltpu.async_copy(bufs[b], out_h.at[c, wb_slice(i)], ssems[b])
        for i in (nwb - 2, nwb - 1):
            pltpu.make_async_copy(bufs[i % 2], out_h.at[c, wb_slice(i)],
                                  ssems[i % 2]).wait()

    return k(h, src_flat, dst_flat, nrm_flat)


def _pair_products(h, ps, pd, ns, nd):
    P = ps.shape[0]
    C = 80
    total = P // C
    iters = (total + _NW - 1) // _NW

    @functools.partial(
        pl.kernel,
        compiler_params=pltpu.CompilerParams(needs_layout_passes=False),
        out_type=jax.ShapeDtypeStruct((2 * P, _D), jnp.float32),
        mesh=_sc_mesh(),
        scratch_types=[
            pltpu.VMEM((C,), jnp.int32),
            pltpu.VMEM((C,), jnp.int32),
            pltpu.VMEM((C, _D), jnp.float32),
            pltpu.VMEM((C, _D), jnp.float32),
            pltpu.SemaphoreType.DMA,
        ],
    )
    def k(h_h, ps_h, pd_h, ns_h, nd_h, z_h, a_v, b_v, ra_v, rb_v, sem):
        c = lax.axis_index("c")
        s = lax.axis_index("s")
        wid = c * _NS + s

        def do(pa_h, pb_h, obase):
            def chunk(t, carry):
                ci = wid + t * _NW

                @pl.when(ci < total)
                def _():
                    cb = ci * C
                    pltpu.sync_copy(pa_h.at[pl.ds(cb, C)], a_v)
                    pltpu.sync_copy(pb_h.at[pl.ds(cb, C)], b_v)
                    pltpu.async_copy(h_h.at[a_v], ra_v, sem).wait()
                    pltpu.async_copy(h_h.at[b_v], rb_v, sem).wait()

                    def mul(e, carry2):
                        for j in range(_DG):
                            sl = pl.ds(j * _L, _L)
                            ra_v[e, sl] = ra_v[e, sl] * rb_v[e, sl]
                        return carry2

                    lax.fori_loop(0, C, mul, 0)
                    pltpu.sync_copy(ra_v, z_h.at[pl.ds(obase + cb, C)])

                return carry

            lax.fori_loop(0, iters, chunk, 0)

        do(ps_h, pd_h, 0)
        do(ns_h, nd_h, P)

    return k(h, ps, pd, ns, nd)


def _add_body(a_ref, b_ref, o_ref):
    o_ref[...] = a_ref[0] + b_ref[0]


def _combine(p):
    N = p.shape[1]
    blk = 2048
    return pl.pallas_call(
        _add_body,
        grid=(N // blk,),
        in_specs=[pl.BlockSpec((1, blk, _D), lambda i: (0, i, 0)),
                  pl.BlockSpec((1, blk, _D), lambda i: (1, i, 0))],
        out_specs=pl.BlockSpec((blk, _D), lambda i: (i, 0)),
        out_shape=jax.ShapeDtypeStruct((N, _D), jnp.float32),
    )(p, p)


def _tag_linear_body(h0, h1, h2, p3a, p3b, w_ref, b_ref, o_ref, *, relu):
    w = w_ref[...]
    acc = (h0[...] @ w[0:128]
           + h1[...] @ w[128:256]
           + h2[...] @ w[256:384]
           + (p3a[0] + p3b[0]) @ w[384:512]
           + b_ref[...])
    o_ref[...] = jnp.maximum(acc, 0.0) if relu else acc


def _tag_linear(h0, h1, h2, p3, W, b, relu):
    N = h0.shape[0]
    blk = 2048
    return pl.pallas_call(
        functools.partial(_tag_linear_body, relu=relu),
        grid=(N // blk,),
        in_specs=[pl.BlockSpec((blk, _D), lambda i: (i, 0))] * 3
        + [pl.BlockSpec((1, blk, _D), lambda i: (0, i, 0)),
           pl.BlockSpec((1, blk, _D), lambda i: (1, i, 0)),
           pl.BlockSpec((4 * _D, _D), lambda i: (0, 0)),
           pl.BlockSpec((_D,), lambda i: (0,))],
        out_specs=pl.BlockSpec((blk, _D), lambda i: (i, 0)),
        out_shape=jax.ShapeDtypeStruct((N, _D), jnp.float32),
    )(h0, h1, h2, p3, p3, W, b)


def _pred_body(z_ref, p1_ref, pb1_ref, p2_ref, pb2_ref, p3_ref, pb3_ref, o_ref):
    t = z_ref[...] @ p1_ref[...] + pb1_ref[...]
    t = jnp.where(t > 0, t, 0.2 * t)
    t = t @ p2_ref[...] + pb2_ref[...]
    t = jnp.where(t > 0, t, 0.2 * t)
    o_ref[...] = t @ p3_ref[...] + pb3_ref[...]


def _predictor(z, P1, pb1, P2, pb2, P3, pb3):
    B = z.shape[0]
    blk = 2000
    return pl.pallas_call(
        _pred_body,
        grid=(B // blk,),
        in_specs=[
            pl.BlockSpec((blk, _D), lambda i: (i, 0)),
            pl.BlockSpec((_D, 64), lambda i: (0, 0)),
            pl.BlockSpec((64,), lambda i: (0,)),
            pl.BlockSpec((64, 32), lambda i: (0, 0)),
            pl.BlockSpec((32,), lambda i: (0,)),
            pl.BlockSpec((32, 1), lambda i: (0, 0)),
            pl.BlockSpec((1,), lambda i: (0,)),
        ],
        out_specs=pl.BlockSpec((blk, 1), lambda i: (i, 0)),
        out_shape=jax.ShapeDtypeStruct((B, 1), jnp.float32),
    )(z, P1, pb1, P2, pb2, P3, pb3)


def kernel(x, edge_index, edge_weight, pos_edges, neg_edges,
           W1, b1, W2, b2, W3, b3, P1, pb1, P2, pb2, P3, pb3):
    N = x.shape[0]
    n_pad = ((N + _NW * _L - 1) // (_NW * _L)) * (_NW * _L)
    src = edge_index[0].astype(jnp.int32)
    dst = edge_index[1].astype(jnp.int32)
    w = edge_weight.astype(jnp.float32)

    parts = _deg_partials(src, dst, w, n_pad)
    norm = _edge_norm(parts, src, dst, w, n_pad)


    h = jnp.pad(x, ((0, n_pad - N), (0, 0)))
    for W, b, act in ((W1, b1, True), (W2, b2, True), (W3, b3, False)):
        f0 = h
        p1 = _prop(f0, src, dst, norm)
        h1 = _combine(p1)
        p2 = _prop(h1, src, dst, norm)
        h2 = _combine(p2)
        p3 = _prop(h2, src, dst, norm)
        h = _tag_linear(f0, h1, h2, p3, W, b, act)

    pe = pos_edges.astype(jnp.int32)
    ne = neg_edges.astype(jnp.int32)
    z = _pair_products(h, pe[0], pe[1], ne[0], ne[1])
    hz = _predictor(z, P1, pb1, P2, pb2, P3, pb3)
    P = pe.shape[1]
    return (hz[:P], hz[P:], h[:N])
